# Initial kernel scaffold; baseline (speedup 1.0000x reference)
import jax
import jax.numpy as jnp
from jax.experimental import pallas as pl


def kernel(x, edge_index, edge_attr, weight, root, bias):
    raise NotImplementedError("write your pallas kernel here")


# speedup vs baseline: 3.0914x; 3.0914x over previous
import functools

import jax
import jax.numpy as jnp
from jax import lax
from jax.experimental import pallas as pl
from jax.experimental.pallas import tpu as pltpu
from jax.experimental.pallas import tpu_sc as plsc

NC = 2
NS = 16
LANES = 16
CHUNK = 128


def _sc_accumulate(xs, src_p, dst_p, v_p, zrow, *, n_rows, w, ept):
    rpt = n_rows // NS
    nchunks = ept // CHUNK
    n_half = xs.shape[0] // 2

    mesh = plsc.VectorSubcoreMesh(
        core_axis_name="c", subcore_axis_name="s",
        num_cores=NC, num_subcores=NS)

    @functools.partial(
        pl.kernel,
        out_type=(jax.ShapeDtypeStruct((NC * n_rows, w), jnp.float32),
                  jax.ShapeDtypeStruct((NC * n_rows, w), jnp.float32)),
        mesh=mesh,
        scratch_types=[
            pltpu.VMEM((CHUNK,), jnp.int32),
            pltpu.VMEM((CHUNK,), jnp.int32),
            pltpu.VMEM((CHUNK + LANES,), jnp.float32),
            pltpu.VMEM((CHUNK, w), jnp.float32),
            pltpu.VMEM_SHARED((n_rows, w), jnp.float32),
            pltpu.VMEM_SHARED((n_rows, w), jnp.float32),
            pltpu.SemaphoreType.DMA,
        ],
        compiler_params=pltpu.CompilerParams(use_tc_tiling_on_sc=False),
    )
    def body(xs_hbm, src_hbm, dst_hbm, v_hbm, zrow_hbm, sall_hbm, s1_hbm,
             src_v, dst_v, vv, rows, acc_all, acc_s1, sem):
        c = lax.axis_index("c")
        s = lax.axis_index("s")

        pltpu.sync_copy(zrow_hbm, acc_all.at[pl.ds(s * rpt, rpt)])
        pltpu.sync_copy(zrow_hbm, acc_s1.at[pl.ds(s * rpt, rpt)])
        plsc.subcore_barrier()

        ebase = s * ept
        goff = c * n_half

        def chunk_body(i, carry):
            b = ebase + i * CHUNK
            pltpu.sync_copy(src_hbm.at[pl.ds(b, CHUNK)], src_v)
            pltpu.sync_copy(dst_hbm.at[pl.ds(b, CHUNK)], dst_v)
            pltpu.sync_copy(v_hbm.at[pl.ds(b, CHUNK)], vv.at[pl.ds(0, CHUNK)])
            for g in range(CHUNK // LANES):
                sl = pl.ds(g * LANES, LANES)
                src_v[sl] = src_v[sl] + goff
            pltpu.async_copy(xs_hbm.at[src_v], rows, sem).wait()
            pltpu.sync_copy(rows, acc_all.at[dst_v], add=True)

            def escale(e, cc):
                vs = vv[pl.ds(e, LANES)][0]
                for j in range(w // LANES):
                    sl = pl.ds(j * LANES, LANES)
                    rows[e, sl] = rows[e, sl] * vs
                return cc

            lax.fori_loop(0, CHUNK, escale, 0)
            pltpu.sync_copy(rows, acc_s1.at[dst_v], add=True)
            return carry

        lax.fori_loop(0, nchunks, chunk_body, 0)
        plsc.subcore_barrier()

        ob = c * n_rows + s * rpt
        pltpu.sync_copy(acc_all.at[pl.ds(s * rpt, rpt)],
                        sall_hbm.at[pl.ds(ob, rpt)])
        pltpu.sync_copy(acc_s1.at[pl.ds(s * rpt, rpt)],
                        s1_hbm.at[pl.ds(ob, rpt)])

    return body(xs, src_p, dst_p, v_p, zrow)


def kernel(x, edge_index, edge_attr, weight, root, bias):
    n, d = x.shape
    e = edge_index.shape[1]
    h = d // 2
    w = 80

    blk = 512
    n_rows = ((n + 1 + blk - 1) // blk) * blk
    nblocks = n_rows // blk

    ept = -(-e // (NS * CHUNK)) * CHUNK
    e_pad = ept * NS
    pad = e_pad - e

    src = edge_index[0].astype(jnp.int32)
    dst = edge_index[1].astype(jnp.int32)
    val = edge_attr[:, 0].astype(jnp.float32)
    src_p = jnp.pad(src, (0, pad))
    dst_p = jnp.pad(dst, (0, pad), constant_values=n)
    v_p = jnp.pad(val, (0, pad))
    xs0 = jnp.concatenate([x[:, :h], jnp.zeros((n, w - h), x.dtype)], axis=1)
    xs1 = jnp.concatenate([x[:, h:], jnp.ones((n, 1), x.dtype),
                           jnp.zeros((n, w - h - 1), x.dtype)], axis=1)
    xs = jnp.concatenate([xs0, xs1], axis=0)
    zrow = jnp.zeros((n_rows // NS, w), jnp.float32)

    sall, s1 = _sc_accumulate(xs, src_p, dst_p, v_p, zrow,
                              n_rows=n_rows, w=w, ept=ept)

    def tc_body(sa0, sa1, sb0, sb1, xr, wr, rr, br, outr):
        sall_b = jnp.concatenate([sa0[:, :h], sa1[:, :h]], axis=1)
        s1_b = jnp.concatenate([sb0[:, :h], sb1[:, :h]], axis=1)
        cnt = sa1[:, h:h + 1]
        w0 = wr[0]
        w10 = wr[1] - wr[0]
        pre = (jnp.dot(sall_b, w0, preferred_element_type=jnp.float32)
               + jnp.dot(s1_b, w10, preferred_element_type=jnp.float32))
        agg = pre / jnp.maximum(cnt, 1.0)
        y = (agg + jnp.dot(xr[...], rr[...],
                           preferred_element_type=jnp.float32) + br[...])
        outr[...] = jnp.maximum(y, 0.0)

    x_pad = jnp.concatenate(
        [x, jnp.zeros((n_rows - n, d), x.dtype)], axis=0)
    y = pl.pallas_call(
        tc_body,
        grid=(nblocks,),
        in_specs=[
            pl.BlockSpec((blk, w), lambda i: (i, 0)),
            pl.BlockSpec((blk, w), lambda i: (i + nblocks, 0)),
            pl.BlockSpec((blk, w), lambda i: (i, 0)),
            pl.BlockSpec((blk, w), lambda i: (i + nblocks, 0)),
            pl.BlockSpec((blk, d), lambda i: (i, 0)),
            pl.BlockSpec((2, d, d), lambda i: (0, 0, 0)),
            pl.BlockSpec((d, d), lambda i: (0, 0)),
            pl.BlockSpec((1, d), lambda i: (0, 0)),
        ],
        out_specs=pl.BlockSpec((blk, d), lambda i: (i, 0)),
        out_shape=jax.ShapeDtypeStruct((n_rows, d), jnp.float32),
    )(sall, sall, s1, s1, x_pad, weight, root, bias.reshape(1, d))
    return y[:n]


# speedup vs baseline: 3.8811x; 1.2555x over previous
import functools

import jax
import jax.numpy as jnp
from jax import lax
from jax.experimental import pallas as pl
from jax.experimental.pallas import tpu as pltpu
from jax.experimental.pallas import tpu_sc as plsc

NC = 2
NS = 16
LANES = 16
CHUNK = 128


def _sc_accumulate(xs, edata, zrow_a, zrow_b, *, n_rows, w, ws, nchunks):
    rpt = n_rows // NS

    mesh = plsc.VectorSubcoreMesh(
        core_axis_name="c", subcore_axis_name="s",
        num_cores=NC, num_subcores=NS)

    @functools.partial(
        pl.kernel,
        out_type=(jax.ShapeDtypeStruct((NC * n_rows, w), jnp.float32),
                  jax.ShapeDtypeStruct((NC * n_rows, ws), jnp.float32)),
        mesh=mesh,
        scratch_types=[
            pltpu.VMEM((4, CHUNK), jnp.int32),
            pltpu.VMEM((4, CHUNK), jnp.int32),
            pltpu.VMEM((4, CHUNK), jnp.int32),
            pltpu.VMEM((4, CHUNK), jnp.int32),
            pltpu.VMEM((CHUNK, w), jnp.float32),
            pltpu.VMEM((CHUNK, w), jnp.float32),
            pltpu.VMEM((CHUNK, ws), jnp.float32),
            pltpu.VMEM((CHUNK, ws), jnp.float32),
            pltpu.VMEM_SHARED((n_rows, w), jnp.float32),
            pltpu.VMEM_SHARED((n_rows, ws), jnp.float32),
            pltpu.SemaphoreType.DMA,
            pltpu.SemaphoreType.DMA,
            pltpu.SemaphoreType.DMA,
            pltpu.SemaphoreType.DMA,
            pltpu.SemaphoreType.DMA,
            pltpu.SemaphoreType.DMA,
            pltpu.SemaphoreType.DMA,
            pltpu.SemaphoreType.DMA,
            pltpu.SemaphoreType.DMA,
            pltpu.SemaphoreType.DMA,
        ],
        compiler_params=pltpu.CompilerParams(use_tc_tiling_on_sc=False,
                                             needs_layout_passes=False),
    )
    def body(xs_hbm, edata_hbm, zrow_a_hbm, zrow_b_hbm, sall_hbm, s1_hbm,
             ed0, ed1, ed2, ed3, rows0, rows1, srows0, srows1,
             acc_all, acc_s1,
             sem_e0, sem_e1, sem_e2, sem_e3,
             sem_g0, sem_g1, sem_a0, sem_a1, sem_b0, sem_b1):
        c = lax.axis_index("c")
        s = lax.axis_index("s")
        eds = (ed0, ed1, ed2, ed3)
        sem_e = (sem_e0, sem_e1, sem_e2, sem_e3)
        rows = (rows0, rows1)
        srows = (srows0, srows1)
        sem_g = (sem_g0, sem_g1)
        sem_a = (sem_a0, sem_a1)
        sem_b = (sem_b0, sem_b1)

        pltpu.sync_copy(zrow_a_hbm, acc_all.at[pl.ds(s * rpt, rpt)])
        pltpu.sync_copy(zrow_b_hbm, acc_s1.at[pl.ds(s * rpt, rpt)])
        pltpu.sync_copy(edata_hbm.at[c, s, 0], ed0)
        pltpu.async_copy(edata_hbm.at[c, s, 1], ed1, sem_e1)
        plsc.subcore_barrier()

        def gather(islot, sl):
            return pltpu.async_copy(xs_hbm.at[eds[islot].at[0]], rows[sl],
                                    sem_g[sl])

        def scale(islot, sl):
            def ebody(e, cc):
                vs = plsc.bitcast(eds[islot][2, pl.ds(e, LANES)],
                                  jnp.float32)[0]
                for j in range(ws // LANES):
                    csl = pl.ds(j * LANES, LANES)
                    srows[sl][e, csl] = rows[sl][e, csl] * vs
                return cc
            lax.fori_loop(0, CHUNK, ebody, 0)

        gather(0, 0)

        def step(i, j):
            sl = j % 2
            @pl.when(i + 1 < nchunks)
            def _():
                @pl.when(i >= 1)
                def _():
                    pltpu.make_async_copy(
                        rows[1 - sl], acc_all.at[pl.ds(0, CHUNK)],
                        sem_a[1 - sl]).wait()
                pltpu.make_async_copy(
                    edata_hbm.at[c, s, 0], eds[(j + 1) % 4],
                    sem_e[(j + 1) % 4]).wait()
                gather((j + 1) % 4, 1 - sl)
            pltpu.make_async_copy(xs_hbm.at[eds[j].at[0]], rows[sl],
                                  sem_g[sl]).wait()
            pltpu.async_copy(rows[sl], acc_all.at[eds[j].at[1]],
                             sem_a[sl], add=True)
            @pl.when(i >= 2)
            def _():
                pltpu.make_async_copy(
                    srows[sl], acc_s1.at[pl.ds(0, CHUNK)], sem_b[sl]).wait()
            @pl.when(i + 2 < nchunks)
            def _():
                pltpu.async_copy(edata_hbm.at[c, s, i + 2],
                                 eds[(j + 2) % 4], sem_e[(j + 2) % 4])
            scale(j, sl)
            pltpu.async_copy(srows[sl], acc_s1.at[eds[j].at[1]],
                             sem_b[sl], add=True)

        def quad(k, carry):
            for j in range(4):
                step(4 * k + j, j)
            return carry

        lax.fori_loop(0, nchunks // 4, quad, 0)

        pltpu.make_async_copy(rows0, acc_all.at[pl.ds(0, CHUNK)],
                              sem_a0).wait()
        pltpu.make_async_copy(rows1, acc_all.at[pl.ds(0, CHUNK)],
                              sem_a1).wait()
        pltpu.make_async_copy(srows0, acc_s1.at[pl.ds(0, CHUNK)],
                              sem_b0).wait()
        pltpu.make_async_copy(srows1, acc_s1.at[pl.ds(0, CHUNK)],
                              sem_b1).wait()
        plsc.subcore_barrier()

        ob = c * n_rows + s * rpt
        pltpu.sync_copy(acc_all.at[pl.ds(s * rpt, rpt)],
                        sall_hbm.at[pl.ds(ob, rpt)])
        pltpu.sync_copy(acc_s1.at[pl.ds(s * rpt, rpt)],
                        s1_hbm.at[pl.ds(ob, rpt)])

    return body(xs, edata, zrow_a, zrow_b)


def kernel(x, edge_index, edge_attr, weight, root, bias):
    n, d = x.shape
    e = edge_index.shape[1]
    h = d // 2
    w = 80

    blk = 632
    n_rows = ((n + 1 + blk - 1) // blk) * blk
    nblocks = n_rows // blk

    ept = -(-e // (NS * CHUNK * 4)) * (CHUNK * 4)
    e_pad = ept * NS
    pad = e_pad - e

    nchunks = ept // CHUNK
    src = edge_index[0].astype(jnp.int32)
    dst = edge_index[1].astype(jnp.int32)
    val = edge_attr[:, 0].astype(jnp.float32)
    src_p = jnp.pad(src, (0, pad))
    dst_p = jnp.pad(dst, (0, pad), constant_values=n)
    v_p = jnp.pad(val, (0, pad))
    vbits = lax.bitcast_convert_type(v_p, jnp.int32)
    zero_row = jnp.zeros_like(src_p)

    def _plane(a):
        return a.reshape(NS, nchunks, 1, CHUNK)

    edata = jnp.stack([
        jnp.concatenate([_plane(src_p + cc * n), _plane(dst_p),
                         _plane(vbits), _plane(zero_row)], axis=2)
        for cc in range(NC)])
    xs0 = jnp.concatenate([x[:, :h], jnp.zeros((n, w - h), x.dtype)], axis=1)
    xs1 = jnp.concatenate([x[:, h:], jnp.ones((n, 1), x.dtype),
                           jnp.zeros((n, w - h - 1), x.dtype)], axis=1)
    xs = jnp.concatenate([xs0, xs1], axis=0)
    zrow_a = jnp.zeros((n_rows // NS, w), jnp.float32)
    zrow_b = jnp.zeros((n_rows // NS, h), jnp.float32)

    sall, s1 = _sc_accumulate(xs, edata, zrow_a, zrow_b,
                              n_rows=n_rows, w=w, ws=h, nchunks=nchunks)

    def tc_body(sa0, sa1, sb0, sb1, xr, wr, rr, br, outr):
        sall_b = jnp.concatenate([sa0[:, :h], sa1[:, :h]], axis=1)
        s1_b = jnp.concatenate([sb0[...], sb1[...]], axis=1)
        cnt = sa1[:, h:h + 1]
        w0 = wr[0]
        w10 = wr[1] - wr[0]
        pre = (jnp.dot(sall_b, w0, preferred_element_type=jnp.float32)
               + jnp.dot(s1_b, w10, preferred_element_type=jnp.float32))
        agg = pre / jnp.maximum(cnt, 1.0)
        y = (agg + jnp.dot(xr[...], rr[...],
                           preferred_element_type=jnp.float32) + br[...])
        outr[...] = jnp.maximum(y, 0.0)

    x_pad = jnp.concatenate(
        [x, jnp.zeros((n_rows - n, d), x.dtype)], axis=0)
    y = pl.pallas_call(
        tc_body,
        grid=(nblocks,),
        in_specs=[
            pl.BlockSpec((blk, w), lambda i: (i, 0)),
            pl.BlockSpec((blk, w), lambda i: (i + nblocks, 0)),
            pl.BlockSpec((blk, h), lambda i: (i, 0)),
            pl.BlockSpec((blk, h), lambda i: (i + nblocks, 0)),
            pl.BlockSpec((blk, d), lambda i: (i, 0)),
            pl.BlockSpec((2, d, d), lambda i: (0, 0, 0)),
            pl.BlockSpec((d, d), lambda i: (0, 0)),
            pl.BlockSpec((1, d), lambda i: (0, 0)),
        ],
        out_specs=pl.BlockSpec((blk, d), lambda i: (i, 0)),
        out_shape=jax.ShapeDtypeStruct((n_rows, d), jnp.float32),
    )(sall, sall, s1, s1, x_pad, weight, root, bias.reshape(1, d))
    return y[:n]
